# baseline (speedup 1.0000x reference)
import math

import jax
import jax.numpy as jnp
from jax.experimental import pallas as pl
from jax.experimental.pallas import tpu as pltpu

_N, _D, _H, _DH, _KG, _TOPK = 2048, 1024, 16, 64, 64, 32
_QB = 256


def _scores_body(q_ref, k_ref, b_ref, o_ref):
    q = q_ref[0]
    k = k_ref[0]
    s = jax.lax.dot_general(q, k, (((1,), (1,)), ((), ())),
                            preferred_element_type=jnp.float32)
    o_ref[0] = s * 0.125 + b_ref[...]


def kernel(x, bias, norm_w, Wq, Wk, Wv, Wo, Wg, W1, b1, W2, b2, alpha, beta, gamma, delta):
    n, d = _N, _D
    xs = jax.nn.sigmoid(gamma) * x[0]
    xn = xs * jax.lax.rsqrt(jnp.mean(xs * xs, -1, keepdims=True) + 1e-6) * norm_w
    q = (xn @ Wq).reshape(n, _H, _DH).transpose(1, 0, 2)
    k = (xn @ Wk).reshape(n, _H, _DH).transpose(1, 0, 2)
    v = (xn @ Wv).reshape(n, _H, _DH).transpose(1, 0, 2)
    g = xn @ Wg
    gb = jnp.broadcast_to(g[None], (_H, n, _KG))
    qc = jnp.concatenate([q, gb], -1)
    kc = jnp.concatenate([k, gb], -1)
    bias2 = bias.reshape(n, n)

    nqb = n // _QB
    scores = pl.pallas_call(
        _scores_body,
        grid=(nqb, _H),
        in_specs=[
            pl.BlockSpec((1, _QB, 128), lambda i, h: (h, i, 0)),
            pl.BlockSpec((1, n, 128), lambda i, h: (h, 0, 0)),
            pl.BlockSpec((_QB, n), lambda i, h: (i, 0)),
        ],
        out_specs=pl.BlockSpec((1, _QB, n), lambda i, h: (h, i, 0)),
        out_shape=jax.ShapeDtypeStruct((_H, n, n), jnp.float32),
    )(qc, kc, bias2)

    vals, idx = jax.lax.top_k(scores, _TOPK)
    w = jax.nn.softmax(vals, axis=-1)
    v_g = jnp.take_along_axis(v[:, None, :, :], idx[..., None], axis=2)
    attn = jnp.einsum('hnt,hntd->hnd', w, v_g)
    attn = attn.transpose(1, 0, 2).reshape(n, d) @ Wo
    ffn = jax.nn.gelu(xn @ W1 + b1) @ W2 + b2
    out = jax.nn.sigmoid(alpha) * ffn + jax.nn.sigmoid(beta) * attn
    return (out[None], idx[None])


# speedup vs baseline: 11.2897x; 11.2897x over previous
import math

import jax
import jax.numpy as jnp
from jax.experimental import pallas as pl
from jax.experimental.pallas import tpu as pltpu

_N, _D, _H, _DH, _KG, _TOPK = 2048, 1024, 16, 64, 64, 32
_QB = 256
_NEG = -3.0e38


def _attn_body(q_ref, gq_ref, k_ref, gk_ref, v_ref, b_ref, idx_ref, attn_ref,
               s_scr, w_scr):
    q = q_ref[0]
    gq = gq_ref[...]
    k = k_ref[0]
    gk = gk_ref[...]
    dn = (((1,), (1,)), ((), ()))
    s = (jax.lax.dot_general(q, k, dn, preferred_element_type=jnp.float32) * 0.125
         + b_ref[...]
         + jax.lax.dot_general(gq, gk, dn, preferred_element_type=jnp.float32) * 0.125)
    s_scr[...] = s
    w_scr[...] = s
    iota = jax.lax.broadcasted_iota(jnp.int32, (_QB, _N), 1)
    m0 = None
    mt = None
    for t in range(_TOPK):
        w = w_scr[...]
        mt = jnp.max(w, axis=1, keepdims=True)
        ii = jnp.where(w == mt, iota, jnp.int32(_N))
        amin = jnp.min(ii, axis=1, keepdims=True)
        idx_ref[0, :, t:t + 1] = amin
        w_scr[...] = jnp.where(iota == amin, _NEG, w)
        if t == 0:
            m0 = mt
    s = s_scr[...]
    p = jnp.where(s >= mt, jnp.exp(s - m0), 0.0)
    denom = jnp.sum(p, axis=1, keepdims=True)
    a = jax.lax.dot_general(p, v_ref[0], (((1,), (0,)), ((), ())),
                            preferred_element_type=jnp.float32)
    attn_ref[0] = a / denom


def kernel(x, bias, norm_w, Wq, Wk, Wv, Wo, Wg, W1, b1, W2, b2, alpha, beta, gamma, delta):
    n, d = _N, _D
    xs = jax.nn.sigmoid(gamma) * x[0]
    xn = xs * jax.lax.rsqrt(jnp.mean(xs * xs, -1, keepdims=True) + 1e-6) * norm_w
    q = (xn @ Wq).reshape(n, _H, _DH).transpose(1, 0, 2)
    k = (xn @ Wk).reshape(n, _H, _DH).transpose(1, 0, 2)
    v = (xn @ Wv).reshape(n, _H, _DH).transpose(1, 0, 2)
    g = xn @ Wg
    bias2 = bias.reshape(n, n)

    nqb = n // _QB
    idx, attn = pl.pallas_call(
        _attn_body,
        grid=(nqb, _H),
        in_specs=[
            pl.BlockSpec((1, _QB, _DH), lambda i, h: (h, i, 0)),
            pl.BlockSpec((_QB, _KG), lambda i, h: (i, 0)),
            pl.BlockSpec((1, n, _DH), lambda i, h: (h, 0, 0)),
            pl.BlockSpec((n, _KG), lambda i, h: (0, 0)),
            pl.BlockSpec((1, n, _DH), lambda i, h: (h, 0, 0)),
            pl.BlockSpec((_QB, n), lambda i, h: (i, 0)),
        ],
        out_specs=[
            pl.BlockSpec((1, _QB, _TOPK), lambda i, h: (h, i, 0)),
            pl.BlockSpec((1, _QB, _DH), lambda i, h: (h, i, 0)),
        ],
        out_shape=[
            jax.ShapeDtypeStruct((_H, n, _TOPK), jnp.int32),
            jax.ShapeDtypeStruct((_H, n, _DH), jnp.float32),
        ],
        scratch_shapes=[
            pltpu.VMEM((_QB, _N), jnp.float32),
            pltpu.VMEM((_QB, _N), jnp.float32),
        ],
    )(q, g, k, g, v, bias2)

    attn = attn.transpose(1, 0, 2).reshape(n, d) @ Wo
    ffn = jax.nn.gelu(xn @ W1 + b1) @ W2 + b2
    out = jax.nn.sigmoid(alpha) * ffn + jax.nn.sigmoid(beta) * attn
    return (out[None], idx[None])


# speedup vs baseline: 13.3913x; 1.1861x over previous
import math

import jax
import jax.numpy as jnp
from jax.experimental import pallas as pl
from jax.experimental.pallas import tpu as pltpu

_N, _D, _H, _DH, _KG, _TOPK = 2048, 1024, 16, 64, 64, 32
_QB = 256
_NEG = -3.0e38


def _attn_body(q_ref, gq_ref, k_ref, gk_ref, v_ref, b_ref, idx_ref, attn_ref,
               s_scr, w_scr):
    q = q_ref[0]
    gq = gq_ref[...]
    k = k_ref[0]
    gk = gk_ref[...]
    dn = (((1,), (1,)), ((), ()))
    s = (jax.lax.dot_general(q, k, dn, preferred_element_type=jnp.float32) * 0.125
         + b_ref[...]
         + jax.lax.dot_general(gq, gk, dn, preferred_element_type=jnp.float32) * 0.125)
    s_scr[...] = s
    w_scr[...] = s
    iota_f = jax.lax.broadcasted_iota(jnp.int32, (_QB, _N), 1).astype(jnp.float32)
    m0 = None
    mt = None
    for t in range(_TOPK):
        w = w_scr[...]
        mt = jnp.max(w, axis=1, keepdims=True)
        ii = jnp.where(w == mt, iota_f, jnp.float32(_N))
        amin = jnp.min(ii, axis=1, keepdims=True)
        idx_ref[0, :, t:t + 1] = amin.astype(jnp.int32)
        w_scr[...] = jnp.where(ii == amin, _NEG, w)
        if t == 0:
            m0 = mt
    s = s_scr[...]
    p = jnp.where(s >= mt, jnp.exp(s - m0), 0.0)
    denom = jnp.sum(p, axis=1, keepdims=True)
    a = jax.lax.dot_general(p, v_ref[0], (((1,), (0,)), ((), ())),
                            preferred_element_type=jnp.float32)
    attn_ref[0] = a / denom


def kernel(x, bias, norm_w, Wq, Wk, Wv, Wo, Wg, W1, b1, W2, b2, alpha, beta, gamma, delta):
    n, d = _N, _D
    xs = jax.nn.sigmoid(gamma) * x[0]
    xn = xs * jax.lax.rsqrt(jnp.mean(xs * xs, -1, keepdims=True) + 1e-6) * norm_w
    q = (xn @ Wq).reshape(n, _H, _DH).transpose(1, 0, 2)
    k = (xn @ Wk).reshape(n, _H, _DH).transpose(1, 0, 2)
    v = (xn @ Wv).reshape(n, _H, _DH).transpose(1, 0, 2)
    g = xn @ Wg
    bias2 = bias.reshape(n, n)

    nqb = n // _QB
    idx, attn = pl.pallas_call(
        _attn_body,
        grid=(nqb, _H),
        in_specs=[
            pl.BlockSpec((1, _QB, _DH), lambda i, h: (h, i, 0)),
            pl.BlockSpec((_QB, _KG), lambda i, h: (i, 0)),
            pl.BlockSpec((1, n, _DH), lambda i, h: (h, 0, 0)),
            pl.BlockSpec((n, _KG), lambda i, h: (0, 0)),
            pl.BlockSpec((1, n, _DH), lambda i, h: (h, 0, 0)),
            pl.BlockSpec((_QB, n), lambda i, h: (i, 0)),
        ],
        out_specs=[
            pl.BlockSpec((1, _QB, _TOPK), lambda i, h: (h, i, 0)),
            pl.BlockSpec((1, _QB, _DH), lambda i, h: (h, i, 0)),
        ],
        out_shape=[
            jax.ShapeDtypeStruct((_H, n, _TOPK), jnp.int32),
            jax.ShapeDtypeStruct((_H, n, _DH), jnp.float32),
        ],
        scratch_shapes=[
            pltpu.VMEM((_QB, _N), jnp.float32),
            pltpu.VMEM((_QB, _N), jnp.float32),
        ],
    )(q, g, k, g, v, bias2)

    attn = attn.transpose(1, 0, 2).reshape(n, d) @ Wo
    ffn = jax.nn.gelu(xn @ W1 + b1) @ W2 + b2
    out = jax.nn.sigmoid(alpha) * ffn + jax.nn.sigmoid(beta) * attn
    return (out[None], idx[None])


# speedup vs baseline: 22.9037x; 1.7103x over previous
import math

import jax
import jax.numpy as jnp
from jax.experimental import pallas as pl
from jax.experimental.pallas import tpu as pltpu

_N, _D, _H, _DH, _KG, _TOPK = 2048, 1024, 16, 64, 64, 32
_QB = 256
_NEG = -3.0e38
_NV = _N // 128
_R = 5


def _attn_body(q_ref, gq_ref, k_ref, gk_ref, v_ref, b_ref, idx_ref, attn_ref,
               s_scr, vals_scr):
    q = q_ref[0]
    gq = gq_ref[...]
    k = k_ref[0]
    gk = gk_ref[...]
    dn = (((1,), (1,)), ((), ()))
    s = (jax.lax.dot_general(q, k, dn, preferred_element_type=jnp.float32) * 0.125
         + b_ref[...]
         + jax.lax.dot_general(gq, gk, dn, preferred_element_type=jnp.float32) * 0.125)
    s_scr[...] = s

    lane = jax.lax.broadcasted_iota(jnp.int32, (_QB, 128), 1).astype(jnp.float32)

    cols = [s[:, v * 128:(v + 1) * 128] for v in range(_NV)]
    bv, gi = [], []
    for _ in range(_R):
        cm = cols[0]
        for v in range(1, _NV):
            cm = jnp.maximum(cm, cols[v])
        eqs = [cols[v] == cm for v in range(_NV)]
        ai = jnp.where(eqs[_NV - 1], jnp.float32(_NV - 1), jnp.float32(_NV))
        for v in range(_NV - 2, -1, -1):
            ai = jnp.where(eqs[v], jnp.float32(v), ai)
        bv.append(cm)
        gi.append(ai * 128.0 + lane)
        cols = [jnp.where(eqs[v], _NEG, cols[v]) for v in range(_NV)]
    cm = cols[0]
    for v in range(1, _NV):
        cm = jnp.maximum(cm, cols[v])
    m_rem = jnp.max(cm, axis=1, keepdims=True)

    v31 = None
    for t in range(_TOPK):
        m = bv[0]
        for j in range(1, _R):
            m = jnp.maximum(m, bv[j])
        mt = jnp.max(m, axis=1, keepdims=True)
        ii = [jnp.where(bv[j] == mt, gi[j], jnp.float32(2 * _N)) for j in range(_R)]
        am = ii[0]
        for j in range(1, _R):
            am = jnp.minimum(am, ii[j])
        amin = jnp.min(am, axis=1, keepdims=True)
        idx_ref[0, :, t:t + 1] = amin.astype(jnp.int32)
        vals_scr[:, t:t + 1] = mt
        bv = [jnp.where(ii[j] == amin, _NEG, bv[j]) for j in range(_R)]
        v31 = mt

    @pl.when(jnp.any(v31 <= m_rem))
    def _fallback():
        iota_f = jax.lax.broadcasted_iota(jnp.int32, (_QB, _N), 1).astype(jnp.float32)
        w = s_scr[...]
        for t in range(_TOPK):
            mt = jnp.max(w, axis=1, keepdims=True)
            ii = jnp.where(w == mt, iota_f, jnp.float32(2 * _N))
            amin = jnp.min(ii, axis=1, keepdims=True)
            idx_ref[0, :, t:t + 1] = amin.astype(jnp.int32)
            vals_scr[:, t:t + 1] = mt
            w = jnp.where(ii == amin, _NEG, w)

    vals = vals_scr[...]
    m0 = vals[:, 0:1]
    thr = vals[:, _TOPK - 1:_TOPK]
    s = s_scr[...]
    p = jnp.where(s >= thr, jnp.exp(s - m0), 0.0)
    denom = jnp.sum(p, axis=1, keepdims=True)
    a = jax.lax.dot_general(p, v_ref[0], (((1,), (0,)), ((), ())),
                            preferred_element_type=jnp.float32)
    attn_ref[0] = a / denom


def kernel(x, bias, norm_w, Wq, Wk, Wv, Wo, Wg, W1, b1, W2, b2, alpha, beta, gamma, delta):
    n, d = _N, _D
    xs = jax.nn.sigmoid(gamma) * x[0]
    xn = xs * jax.lax.rsqrt(jnp.mean(xs * xs, -1, keepdims=True) + 1e-6) * norm_w
    q = (xn @ Wq).reshape(n, _H, _DH).transpose(1, 0, 2)
    k = (xn @ Wk).reshape(n, _H, _DH).transpose(1, 0, 2)
    v = (xn @ Wv).reshape(n, _H, _DH).transpose(1, 0, 2)
    g = xn @ Wg
    bias2 = bias.reshape(n, n)

    nqb = n // _QB
    idx, attn = pl.pallas_call(
        _attn_body,
        grid=(nqb, _H),
        in_specs=[
            pl.BlockSpec((1, _QB, _DH), lambda i, h: (h, i, 0)),
            pl.BlockSpec((_QB, _KG), lambda i, h: (i, 0)),
            pl.BlockSpec((1, n, _DH), lambda i, h: (h, 0, 0)),
            pl.BlockSpec((n, _KG), lambda i, h: (0, 0)),
            pl.BlockSpec((1, n, _DH), lambda i, h: (h, 0, 0)),
            pl.BlockSpec((_QB, n), lambda i, h: (i, 0)),
        ],
        out_specs=[
            pl.BlockSpec((1, _QB, _TOPK), lambda i, h: (h, i, 0)),
            pl.BlockSpec((1, _QB, _DH), lambda i, h: (h, i, 0)),
        ],
        out_shape=[
            jax.ShapeDtypeStruct((_H, n, _TOPK), jnp.int32),
            jax.ShapeDtypeStruct((_H, n, _DH), jnp.float32),
        ],
        scratch_shapes=[
            pltpu.VMEM((_QB, _N), jnp.float32),
            pltpu.VMEM((_QB, _TOPK), jnp.float32),
        ],
    )(q, g, k, g, v, bias2)

    attn = attn.transpose(1, 0, 2).reshape(n, d) @ Wo
    ffn = jax.nn.gelu(xn @ W1 + b1) @ W2 + b2
    out = jax.nn.sigmoid(alpha) * ffn + jax.nn.sigmoid(beta) * attn
    return (out[None], idx[None])
